# Initial kernel scaffold; baseline (speedup 1.0000x reference)
import jax
import jax.numpy as jnp
from jax import lax
from jax.experimental import pallas as pl
from jax.experimental.pallas import tpu as pltpu
from jax.experimental.pallas import tpu_sc as plsc

_NC, _NS = 2, 16
_K = 80


def _deg_body(row2d, zeros16, ones16, deg_out, rowidx_v, ones_v, deg_sh):
    n = deg_sh.shape[0]
    npt = n // _NS
    c = lax.axis_index("c")
    s = lax.axis_index("s")
    tile = c * _NS + s
    ch = rowidx_v.shape[0]
    pltpu.sync_copy(zeros16, deg_sh.at[pl.ds(s * npt, npt)])
    pltpu.sync_copy(row2d.at[pl.ds(tile * ch, ch)], rowidx_v)
    pltpu.sync_copy(ones16, ones_v)
    plsc.subcore_barrier()

    def body(j, carry):
        pltpu.sync_copy(ones_v, deg_sh.at[rowidx_v.at[j]], add=True)
        return carry

    lax.fori_loop(0, ch, body, 0)
    plsc.subcore_barrier()
    pltpu.sync_copy(deg_sh.at[pl.ds(s * npt, npt)],
                    deg_out.at[pl.ds(c * n + s * npt, npt)])


def _scat_body(g_hbm, row2d, col2d, ea, zrows, s_out,
               rowidx_v, colidx_v, w_v, rows_v, s_sh):
    n = s_sh.shape[0]
    npt = n // _NS
    c = lax.axis_index("c")
    s = lax.axis_index("s")
    tile = c * _NS + s
    ch = rowidx_v.shape[0]
    ept = ch * _K
    nf = rows_v.shape[1] // 16
    pltpu.sync_copy(zrows, s_sh.at[pl.ds(s * npt, npt)])
    pltpu.sync_copy(row2d.at[pl.ds(tile * ch, ch)], rowidx_v)
    pltpu.sync_copy(col2d.at[pl.ds(tile * ch, ch)], colidx_v)
    pltpu.sync_copy(ea.at[pl.ds(tile * ept, ept)], w_v)
    plsc.subcore_barrier()

    def body(j, carry):
        pltpu.sync_copy(g_hbm.at[rowidx_v.at[j]], rows_v)
        for k in range(_K):
            w = w_v[j * _K + k]
            for f in range(nf):
                sl = pl.ds(f * 16, 16)
                rows_v[k, sl] = rows_v[k, sl] * w
        pltpu.sync_copy(rows_v, s_sh.at[colidx_v.at[j]], add=True)
        return carry

    lax.fori_loop(0, ch, body, 0)
    plsc.subcore_barrier()
    pltpu.sync_copy(s_sh.at[pl.ds(s * npt, npt)],
                    s_out.at[pl.ds(c * n + s * npt, npt)])


def _mlp_body(x_ref, dp_ref, w1_ref, b1_ref, w2_ref, b2_ref, g_ref):
    h = jnp.maximum(
        jnp.dot(x_ref[...], w1_ref[...], preferred_element_type=jnp.float32)
        + b1_ref[...], 0.0)
    h = jnp.dot(h, w2_ref[...], preferred_element_type=jnp.float32) + b2_ref[...]
    deg = dp_ref[0, :, 0:1] + dp_ref[1, :, 0:1] + 1.0
    g_ref[...] = h * lax.rsqrt(deg)


def _final_body(sp_ref, g_ref, dp_ref, b2d_ref, ep_ref,
                w3_ref, b3_ref, w4_ref, b4_ref, out_ref):
    deg = dp_ref[0, :, 0:1] + dp_ref[1, :, 0:1] + 1.0
    s = sp_ref[0] + sp_ref[1] + g_ref[...]
    r = jnp.maximum(s * lax.rsqrt(deg), 0.0)
    ng = ep_ref.shape[0]
    a = (b2d_ref[...] == lax.broadcasted_iota(jnp.int32, (1, ng), 1))
    a = a.astype(jnp.float32)
    n = a.shape[0]
    counts = lax.dot_general(a, jnp.ones((n, 1), jnp.float32),
                             (((0,), (0,)), ((), ())),
                             preferred_element_type=jnp.float32)
    pooled = lax.dot_general(a, r, (((0,), (0,)), ((), ())),
                             preferred_element_type=jnp.float32)
    pooled = pooled / jnp.maximum(counts, 1.0)
    z = jnp.concatenate([pooled, ep_ref[...]], axis=1)
    z = jnp.maximum(
        jnp.dot(z, w3_ref[...], preferred_element_type=jnp.float32)
        + b3_ref[...], 0.0)
    out_ref[...] = (
        jnp.dot(z, w4_ref[...], preferred_element_type=jnp.float32)
        + b4_ref[0, 0])


def kernel(x, edge_index, edge_attr, batch, energy,
           W1, b1, W2, b2, W3, b3, W4, b4):
    n, d = x.shape
    e = edge_index.shape[1]
    g_cnt = energy.shape[0]
    h_dim = W1.shape[1]
    npt = n // _NS
    ch = e // (_NC * _NS * _K)

    row2d = edge_index[0].reshape(e // _K, _K)
    col2d = edge_index[1].reshape(e // _K, _K)

    mesh = plsc.VectorSubcoreMesh(core_axis_name="c", subcore_axis_name="s",
                                  num_cores=_NC, num_subcores=_NS)

    deg_flat = pl.kernel(
        _deg_body,
        out_type=jax.ShapeDtypeStruct((_NC * n, 16), jnp.float32),
        mesh=mesh,
        scratch_types=[
            pltpu.VMEM((ch, _K), jnp.int32),
            pltpu.VMEM((_K, 16), jnp.float32),
            pltpu.VMEM_SHARED((n, 16), jnp.float32),
        ],
    )(row2d, jnp.zeros((npt, 16), jnp.float32), jnp.ones((_K, 16), jnp.float32))
    dp3 = deg_flat.reshape(_NC, n, 16)

    nb = n // 8
    g = pl.pallas_call(
        _mlp_body,
        out_shape=jax.ShapeDtypeStruct((n, d), jnp.float32),
        grid=(8,),
        in_specs=[
            pl.BlockSpec((nb, d), lambda i: (i, 0)),
            pl.BlockSpec((_NC, nb, 16), lambda i: (0, i, 0)),
            pl.BlockSpec((d, h_dim), lambda i: (0, 0)),
            pl.BlockSpec((1, h_dim), lambda i: (0, 0)),
            pl.BlockSpec((h_dim, d), lambda i: (0, 0)),
            pl.BlockSpec((1, d), lambda i: (0, 0)),
        ],
        out_specs=pl.BlockSpec((nb, d), lambda i: (i, 0)),
    )(x, dp3, W1, b1[None], W2, b2[None])

    s_flat = pl.kernel(
        _scat_body,
        out_type=jax.ShapeDtypeStruct((_NC * n, d), jnp.float32),
        mesh=mesh,
        scratch_types=[
            pltpu.VMEM((ch, _K), jnp.int32),
            pltpu.VMEM((ch, _K), jnp.int32),
            pltpu.VMEM((ch * _K,), jnp.float32),
            pltpu.VMEM((_K, d), jnp.float32),
            pltpu.VMEM_SHARED((n, d), jnp.float32),
        ],
    )(g, row2d, col2d, edge_attr, jnp.zeros((npt, d), jnp.float32))
    sp3 = s_flat.reshape(_NC, n, d)

    p = 2 * d
    pad_z = p - (d + energy.shape[1])
    w3p = jnp.pad(W3, ((0, pad_z), (0, pad_z)))
    b3p = jnp.pad(b3, (0, pad_z))[None]
    w4p = jnp.pad(W4, ((0, pad_z), (0, d - W4.shape[1])))
    ep = jnp.pad(energy, ((0, 0), (0, d - energy.shape[1])))

    z = pl.pallas_call(
        _final_body,
        out_shape=jax.ShapeDtypeStruct((g_cnt, d), jnp.float32),
    )(sp3, g, dp3, batch[:, None], ep, w3p, b3p, w4p, b4.reshape(1, 1))
    return z[:, :1]


# speedup vs baseline: 24.0412x; 24.0412x over previous
import jax
import jax.numpy as jnp
from jax import lax
from jax.experimental import pallas as pl
from jax.experimental.pallas import tpu as pltpu
from jax.experimental.pallas import tpu_sc as plsc

_NC, _NS = 2, 16
_K = 80
_CHP = 64


def _deg_body(row2d, hist_out, rowidx_v, hist_v):
    n = hist_v.shape[0]
    c = lax.axis_index("c")
    s = lax.axis_index("s")
    tile = c * _NS + s
    ept = rowidx_v.shape[0]
    pltpu.sync_copy(row2d.at[tile], rowidx_v)

    def zit(i, carry):
        hist_v[pl.ds(i * 16, 16)] = jnp.zeros((16,), jnp.float32)
        return carry

    lax.fori_loop(0, n // 16, zit, 0)
    ones16 = jnp.ones((16,), jnp.float32)

    def it(j, carry):
        idx = rowidx_v[pl.ds(j * 16, 16)]
        plsc.addupdate_scatter(hist_v, [idx], ones16)
        return carry

    lax.fori_loop(0, ept // 16, it, 0)
    pltpu.sync_copy(hist_v, hist_out.at[tile])


def _scat_body(g_hbm, row3d, col3d, ea, zfull, s_out,
               rowidx_v, colidx_v, w_v, rows_v, s_sh):
    c = lax.axis_index("c")
    s = lax.axis_index("s")
    tile = c * _NS + s
    ch = row3d.shape[1]
    ept = ch * _K
    nf = rows_v.shape[1] // 16

    @pl.when(s == 0)
    def _():
        pltpu.sync_copy(zfull, s_sh)

    plsc.subcore_barrier()

    def body(j, carry):
        pltpu.sync_copy(g_hbm.at[rowidx_v.at[j]], rows_v)
        for kb in range(_K // 16):
            wv = w_v[pl.ds(j * _K + kb * 16, 16)]
            for kk in range(16):
                w = wv[kk]
                k = kb * 16 + kk
                for f in range(nf):
                    sl = pl.ds(f * 16, 16)
                    rows_v[k, sl] = rows_v[k, sl] * w
        pltpu.sync_copy(rows_v, s_sh.at[colidx_v.at[j]], add=True)
        return carry

    for p0 in range(0, ch, _CHP):
        pch = min(_CHP, ch - p0)
        pltpu.sync_copy(row3d.at[tile, pl.ds(p0, pch)],
                        rowidx_v.at[pl.ds(0, pch)])
        pltpu.sync_copy(col3d.at[tile, pl.ds(p0, pch)],
                        colidx_v.at[pl.ds(0, pch)])
        pltpu.sync_copy(ea.at[pl.ds(tile * ept + p0 * _K, pch * _K)],
                        w_v.at[pl.ds(0, pch * _K)])
        lax.fori_loop(0, pch, body, 0)

    plsc.subcore_barrier()

    @pl.when(s == 0)
    def _():
        pltpu.sync_copy(s_sh, s_out.at[c])


def _mlp_body(x_ref, dp_ref, w1_ref, b1_ref, w2_ref, b2_ref, g_ref):
    h = jnp.maximum(
        jnp.dot(x_ref[...], w1_ref[...], preferred_element_type=jnp.float32)
        + b1_ref[...], 0.0)
    h = jnp.dot(h, w2_ref[...], preferred_element_type=jnp.float32) + b2_ref[...]
    nt = dp_ref.shape[0]
    deg = lax.dot_general(dp_ref[...], jnp.ones((nt, 1), jnp.float32),
                          (((0,), (0,)), ((), ())),
                          preferred_element_type=jnp.float32) + 1.0
    g_ref[...] = h * lax.rsqrt(deg)


def _final_body(sp_ref, g_ref, dp_ref, b2d_ref, ep_ref,
                w3_ref, b3_ref, w4_ref, b4_ref, out_ref):
    nt = dp_ref.shape[0]
    deg = lax.dot_general(dp_ref[...], jnp.ones((nt, 1), jnp.float32),
                          (((0,), (0,)), ((), ())),
                          preferred_element_type=jnp.float32) + 1.0
    s = sp_ref[0] + sp_ref[1] + g_ref[...]
    r = jnp.maximum(s * lax.rsqrt(deg), 0.0)
    ng = ep_ref.shape[0]
    a = (b2d_ref[...] == lax.broadcasted_iota(jnp.int32, (1, ng), 1))
    a = a.astype(jnp.float32)
    n = a.shape[0]
    counts = lax.dot_general(a, jnp.ones((n, 1), jnp.float32),
                             (((0,), (0,)), ((), ())),
                             preferred_element_type=jnp.float32)
    pooled = lax.dot_general(a, r, (((0,), (0,)), ((), ())),
                             preferred_element_type=jnp.float32)
    pooled = pooled / jnp.maximum(counts, 1.0)
    z = jnp.concatenate([pooled, ep_ref[...]], axis=1)
    z = jnp.maximum(
        jnp.dot(z, w3_ref[...], preferred_element_type=jnp.float32)
        + b3_ref[...], 0.0)
    out_ref[...] = (
        jnp.dot(z, w4_ref[...], preferred_element_type=jnp.float32)
        + b4_ref[0, 0])


def kernel(x, edge_index, edge_attr, batch, energy,
           W1, b1, W2, b2, W3, b3, W4, b4):
    n, d = x.shape
    e = edge_index.shape[1]
    g_cnt = energy.shape[0]
    h_dim = W1.shape[1]
    nt = _NC * _NS
    ept = e // nt
    ch = ept // _K

    row3d = edge_index[0].reshape(nt, ch, _K)
    col3d = edge_index[1].reshape(nt, ch, _K)

    mesh = plsc.VectorSubcoreMesh(core_axis_name="c", subcore_axis_name="s",
                                  num_cores=_NC, num_subcores=_NS)

    hist = pl.kernel(
        _deg_body,
        out_type=jax.ShapeDtypeStruct((nt, n), jnp.float32),
        mesh=mesh,
        compiler_params=pltpu.CompilerParams(needs_layout_passes=False),
        scratch_types=[
            pltpu.VMEM((ept,), jnp.int32),
            pltpu.VMEM((n,), jnp.float32),
        ],
    )(edge_index[0].reshape(nt, ept))

    g = pl.pallas_call(
        _mlp_body,
        out_shape=jax.ShapeDtypeStruct((n, d), jnp.float32),
    )(x, hist, W1, b1[None], W2, b2[None])

    sp = pl.kernel(
        _scat_body,
        out_type=jax.ShapeDtypeStruct((_NC, n, d), jnp.float32),
        mesh=mesh,
        scratch_types=[
            pltpu.VMEM((_CHP, _K), jnp.int32),
            pltpu.VMEM((_CHP, _K), jnp.int32),
            pltpu.VMEM((_CHP * _K,), jnp.float32),
            pltpu.VMEM((_K, d), jnp.float32),
            pltpu.VMEM_SHARED((n, d), jnp.float32),
        ],
    )(g, row3d, col3d, edge_attr, jnp.zeros((n, d), jnp.float32))

    p = 2 * d
    pad_z = p - (d + energy.shape[1])
    w3p = jnp.pad(W3, ((0, pad_z), (0, pad_z)))
    b3p = jnp.pad(b3, (0, pad_z))[None]
    w4p = jnp.pad(W4, ((0, pad_z), (0, d - W4.shape[1])))
    ep = jnp.pad(energy, ((0, 0), (0, d - energy.shape[1])))

    z = pl.pallas_call(
        _final_body,
        out_shape=jax.ShapeDtypeStruct((g_cnt, d), jnp.float32),
    )(sp, g, hist, batch[:, None], ep, w3p, b3p, w4p, b4.reshape(1, 1))
    return z[:, :1]


# speedup vs baseline: 33.2825x; 1.3844x over previous
import jax
import jax.numpy as jnp
from jax import lax
from jax.experimental import pallas as pl
from jax.experimental.pallas import tpu as pltpu
from jax.experimental.pallas import tpu_sc as plsc

_NC, _NS = 2, 16
_K = 80
_CHP = 64


def _deg_body(row2d, hist_out, rowidx_v, hist_v):
    n = hist_v.shape[0]
    c = lax.axis_index("c")
    s = lax.axis_index("s")
    tile = c * _NS + s
    ept = rowidx_v.shape[0]
    pltpu.sync_copy(row2d.at[tile], rowidx_v)

    def zit(i, carry):
        hist_v[pl.ds(i * 16, 16)] = jnp.zeros((16,), jnp.float32)
        return carry

    lax.fori_loop(0, n // 16, zit, 0)
    ones16 = jnp.ones((16,), jnp.float32)

    def it(j, carry):
        idx = rowidx_v[pl.ds(j * 16, 16)]
        plsc.addupdate_scatter(hist_v, [idx], ones16)
        return carry

    lax.fori_loop(0, ept // 16, it, 0)
    pltpu.sync_copy(hist_v, hist_out.at[tile])


def _scat_body(g_hbm, row3d, col3d, ea, zfull, s_out,
               rowidx_v, colidx_v, w_v, rows_a, rows_b,
               sga, sgb, ssa, ssb, s_sh):
    c = lax.axis_index("c")
    s = lax.axis_index("s")
    tile = c * _NS + s
    ch = row3d.shape[1]
    ept = ch * _K
    nf = rows_a.shape[1] // 16

    @pl.when(s == 0)
    def _():
        pltpu.sync_copy(zfull, s_sh)

    plsc.subcore_barrier()

    def scale(buf, j):
        for kb in range(_K // 16):
            wv = w_v[pl.ds(j * _K + kb * 16, 16)]
            for kk in range(16):
                w = wv[kk]
                k = kb * 16 + kk
                for f in range(nf):
                    sl = pl.ds(f * 16, 16)
                    buf[k, sl] = buf[k, sl] * w

    def start_gather(buf, sem, j):
        pltpu.async_copy(g_hbm.at[rowidx_v.at[j]], buf, sem)

    def wait_gather(buf, sem):
        pltpu.make_async_copy(g_hbm.at[rowidx_v.at[0]], buf, sem).wait()

    def start_scatter(buf, sem, j):
        pltpu.async_copy(buf, s_sh.at[colidx_v.at[j]], sem, add=True)

    def wait_scatter(buf, sem):
        pltpu.make_async_copy(buf, s_sh.at[colidx_v.at[0]], sem).wait()

    for p0 in range(0, ch, _CHP):
        pch = min(_CHP, ch - p0)
        pltpu.sync_copy(row3d.at[tile, pl.ds(p0, pch)],
                        rowidx_v.at[pl.ds(0, pch)])
        pltpu.sync_copy(col3d.at[tile, pl.ds(p0, pch)],
                        colidx_v.at[pl.ds(0, pch)])
        pltpu.sync_copy(ea.at[pl.ds(tile * ept + p0 * _K, pch * _K)],
                        w_v.at[pl.ds(0, pch * _K)])
        npair = pch // 2
        start_gather(rows_a, sga, 0)

        def pair(i, carry):
            j0 = 2 * i
            j1 = 2 * i + 1
            wait_gather(rows_a, sga)

            @pl.when(i > 0)
            def _():
                wait_scatter(rows_b, ssb)

            start_gather(rows_b, sgb, j1)
            scale(rows_a, j0)
            start_scatter(rows_a, ssa, j0)
            wait_gather(rows_b, sgb)
            wait_scatter(rows_a, ssa)

            @pl.when(j1 + 1 < pch)
            def _():
                start_gather(rows_a, sga, j1 + 1)

            scale(rows_b, j1)
            start_scatter(rows_b, ssb, j1)
            return carry

        lax.fori_loop(0, npair, pair, 0)
        if pch % 2 == 1:
            wait_gather(rows_a, sga)
            scale(rows_a, pch - 1)
            start_scatter(rows_a, ssa, pch - 1)
            wait_scatter(rows_a, ssa)
        if npair > 0:
            wait_scatter(rows_b, ssb)

    plsc.subcore_barrier()

    @pl.when(s == 0)
    def _():
        pltpu.sync_copy(s_sh, s_out.at[c])


def _mlp_body(x_ref, dp_ref, w1_ref, b1_ref, w2_ref, b2_ref, g_ref):
    h = jnp.maximum(
        jnp.dot(x_ref[...], w1_ref[...], preferred_element_type=jnp.float32)
        + b1_ref[...], 0.0)
    h = jnp.dot(h, w2_ref[...], preferred_element_type=jnp.float32) + b2_ref[...]
    nt = dp_ref.shape[0]
    deg = lax.dot_general(dp_ref[...], jnp.ones((nt, 1), jnp.float32),
                          (((0,), (0,)), ((), ())),
                          preferred_element_type=jnp.float32) + 1.0
    g_ref[...] = h * lax.rsqrt(deg)


def _final_body(sp_ref, g_ref, dp_ref, b2d_ref, ep_ref,
                w3_ref, b3_ref, w4_ref, b4_ref, out_ref):
    nt = dp_ref.shape[0]
    deg = lax.dot_general(dp_ref[...], jnp.ones((nt, 1), jnp.float32),
                          (((0,), (0,)), ((), ())),
                          preferred_element_type=jnp.float32) + 1.0
    s = sp_ref[0] + sp_ref[1] + g_ref[...]
    r = jnp.maximum(s * lax.rsqrt(deg), 0.0)
    ng = ep_ref.shape[0]
    a = (b2d_ref[...] == lax.broadcasted_iota(jnp.int32, (1, ng), 1))
    a = a.astype(jnp.float32)
    n = a.shape[0]
    counts = lax.dot_general(a, jnp.ones((n, 1), jnp.float32),
                             (((0,), (0,)), ((), ())),
                             preferred_element_type=jnp.float32)
    pooled = lax.dot_general(a, r, (((0,), (0,)), ((), ())),
                             preferred_element_type=jnp.float32)
    pooled = pooled / jnp.maximum(counts, 1.0)
    z = jnp.concatenate([pooled, ep_ref[...]], axis=1)
    z = jnp.maximum(
        jnp.dot(z, w3_ref[...], preferred_element_type=jnp.float32)
        + b3_ref[...], 0.0)
    out_ref[...] = (
        jnp.dot(z, w4_ref[...], preferred_element_type=jnp.float32)
        + b4_ref[0, 0])


def kernel(x, edge_index, edge_attr, batch, energy,
           W1, b1, W2, b2, W3, b3, W4, b4):
    n, d = x.shape
    e = edge_index.shape[1]
    g_cnt = energy.shape[0]
    h_dim = W1.shape[1]
    nt = _NC * _NS
    ept = e // nt
    ch = ept // _K

    row3d = edge_index[0].reshape(nt, ch, _K)
    col3d = edge_index[1].reshape(nt, ch, _K)

    mesh = plsc.VectorSubcoreMesh(core_axis_name="c", subcore_axis_name="s",
                                  num_cores=_NC, num_subcores=_NS)

    hist = pl.kernel(
        _deg_body,
        out_type=jax.ShapeDtypeStruct((nt, n), jnp.float32),
        mesh=mesh,
        compiler_params=pltpu.CompilerParams(needs_layout_passes=False),
        scratch_types=[
            pltpu.VMEM((ept,), jnp.int32),
            pltpu.VMEM((n,), jnp.float32),
        ],
    )(edge_index[0].reshape(nt, ept))

    g = pl.pallas_call(
        _mlp_body,
        out_shape=jax.ShapeDtypeStruct((n, d), jnp.float32),
    )(x, hist, W1, b1[None], W2, b2[None])

    sp = pl.kernel(
        _scat_body,
        out_type=jax.ShapeDtypeStruct((_NC, n, d), jnp.float32),
        mesh=mesh,
        scratch_types=[
            pltpu.VMEM((_CHP, _K), jnp.int32),
            pltpu.VMEM((_CHP, _K), jnp.int32),
            pltpu.VMEM((_CHP * _K,), jnp.float32),
            pltpu.VMEM((_K, d), jnp.float32),
            pltpu.VMEM((_K, d), jnp.float32),
            pltpu.SemaphoreType.DMA,
            pltpu.SemaphoreType.DMA,
            pltpu.SemaphoreType.DMA,
            pltpu.SemaphoreType.DMA,
            pltpu.VMEM_SHARED((n, d), jnp.float32),
        ],
    )(g, row3d, col3d, edge_attr, jnp.zeros((n, d), jnp.float32))

    p = 2 * d
    pad_z = p - (d + energy.shape[1])
    w3p = jnp.pad(W3, ((0, pad_z), (0, pad_z)))
    b3p = jnp.pad(b3, (0, pad_z))[None]
    w4p = jnp.pad(W4, ((0, pad_z), (0, d - W4.shape[1])))
    ep = jnp.pad(energy, ((0, 0), (0, d - energy.shape[1])))

    z = pl.pallas_call(
        _final_body,
        out_shape=jax.ShapeDtypeStruct((g_cnt, d), jnp.float32),
    )(sp, g, hist, batch[:, None], ep, w3p, b3p, w4p, b4.reshape(1, 1))
    return z[:, :1]
